# baseline (speedup 1.0000x reference)
import functools

import jax
import jax.numpy as jnp
from jax import lax
from jax.experimental import pallas as pl
from jax.experimental.pallas import tpu as pltpu
from jax.experimental.pallas import tpu_sc as plsc

H = 3
K = 16
OFFSET_SCALE = 10.0

M = 256

_SC_CORES = 2
_SC_SUBCORES = 16
_GATHER_CHUNK = 128

_INV_SQRT2 = 0.7071067811865476


def _gelu_exact(v):
    return 0.5 * v * (1.0 + lax.erf(v * _INV_SQRT2))


def _prep_body(n, coordsT_ref, coords_ref, x_ref, qkvwT_ref, qkvb_ref,
               off1T_ref, off1b_ref, off2T_ref, off2b_ref, w1_ref, b1_ref,
               w2s_ref, pos2b_ref, q_ref, kv_ref, idx_ref, bias_ref):
    xb = x_ref[...]
    qkv = jnp.dot(xb, qkvwT_ref[...], preferred_element_type=jnp.float32)
    qkv = qkv + qkvb_ref[...]
    c = xb.shape[1]
    q_ref[...] = qkv[:, :c]
    kv_ref[...] = qkv[:, c:]

    hid = jnp.dot(xb, off1T_ref[...], preferred_element_type=jnp.float32)
    hid = _gelu_exact(hid + off1b_ref[...])
    offs = jnp.dot(hid, off2T_ref[...], preferred_element_type=jnp.float32)
    offs = offs + off2b_ref[...]

    ph = jnp.dot(offs, w1_ref[...], preferred_element_type=jnp.float32)
    ph = _gelu_exact(ph + b1_ref[...])
    for h in range(H):
        bh = jnp.dot(ph, w2s_ref[h], preferred_element_type=jnp.float32)
        bias_ref[:, K * h:K * (h + 1)] = bh + pos2b_ref[:, h:h + 1]

    ct = coordsT_ref[...]
    cx, cy, cz = ct[0:1, :], ct[1:2, :], ct[2:3, :]
    cb = coords_ref[...]
    iota_f = lax.broadcasted_iota(jnp.int32, (xb.shape[0], n), 1).astype(jnp.float32)
    nn_cols = []
    for k in range(K):
        spx = cb[:, 0:1] + offs[:, 3 * k:3 * k + 1] * OFFSET_SCALE
        spy = cb[:, 1:2] + offs[:, 3 * k + 1:3 * k + 2] * OFFSET_SCALE
        spz = cb[:, 2:3] + offs[:, 3 * k + 2:3 * k + 3] * OFFSET_SCALE
        dx = spx - cx
        dy = spy - cy
        dz = spz - cz
        d2 = dx * dx
        d2 = d2 + dy * dy
        d2 = d2 + dz * dz
        dmin = jnp.min(d2, axis=1, keepdims=True)
        cand = jnp.where(d2 == dmin, iota_f, float(n))
        nn_cols.append(jnp.min(cand, axis=1, keepdims=True))
    nn_mat = jnp.concatenate(nn_cols, axis=1)
    idx_ref[0] = jnp.transpose(nn_mat).astype(jnp.int32)


def _attn_body(q_ref, s_ref, bias_ref, rl_ref, projT_ref, projb_ref, o_ref):
    q = q_ref[...]
    m_, c = q.shape
    d = c // H
    scale = d ** (-0.5)

    dots = []
    for k in range(K):
        prod_qk = s_ref[0, k, :, :c] * q
        dots.append(jnp.dot(prod_qk, rl_ref[k],
                            preferred_element_type=jnp.float32))
    while len(dots) > 1:
        dots = [dots[i] + dots[i + 1] for i in range(0, len(dots), 2)]
    logits = dots[0] * scale + bias_ref[...]

    a_h = []
    for h in range(H):
        lh = logits[:, K * h:K * (h + 1)]
        mx = jnp.max(lh, axis=1, keepdims=True)
        e = jnp.exp(lh - mx)
        a_h.append(e / jnp.sum(e, axis=1, keepdims=True))

    accs = [jnp.zeros((m_, d), jnp.float32) for _ in range(H)]
    for k in range(K):
        for h in range(H):
            svkh = s_ref[0, k, :, c + d * h:c + d * (h + 1)]
            accs[h] = accs[h] + a_h[h][:, k:k + 1] * svkh
    oc = jnp.concatenate(accs, axis=1)
    o_ref[...] = jnp.dot(oc, projT_ref[...], preferred_element_type=jnp.float32) + projb_ref[...]


def _gather_rows(kv_flat, idx_rm):
    nr, mw = idx_rm.shape
    rows, width = nr * mw, kv_flat.shape[1]
    nw = _SC_CORES * _SC_SUBCORES
    per_w = rows // nw
    rows_per_w = per_w // mw
    cols_per_row = mw // _GATHER_CHUNK
    chunks = per_w // _GATHER_CHUNK
    mesh = plsc.VectorSubcoreMesh(core_axis_name="c", subcore_axis_name="s",
                                  num_cores=_SC_CORES, num_subcores=_SC_SUBCORES)

    @functools.partial(
        pl.kernel, mesh=mesh,
        out_type=jax.ShapeDtypeStruct((rows, width), jnp.float32),
        scratch_types=[
            pltpu.VMEM((2, _GATHER_CHUNK), jnp.int32),
            pltpu.VMEM((2, _GATHER_CHUNK, width), jnp.float32),
            [pltpu.SemaphoreType.DMA] * 2,
            [pltpu.SemaphoreType.DMA] * 2,
            [pltpu.SemaphoreType.DMA] * 2,
        ],
    )
    def gather_kernel(kv_hbm, idx_hbm, out_hbm, idx_v, rows_v, semi, semg, semo):
        wid = lax.axis_index("s") * _SC_CORES + lax.axis_index("c")
        base = wid * per_w

        def start_idx(ci):
            p = ci % 2
            row = wid * rows_per_w + ci // cols_per_row
            col = (ci % cols_per_row) * _GATHER_CHUNK
            return pltpu.async_copy(
                idx_hbm.at[row, pl.ds(col, _GATHER_CHUNK)],
                idx_v.at[p], semi[p])

        idx_copies = {0: start_idx(0)}
        out_copies = {}
        for ci in range(chunks):
            p = ci % 2
            if ci >= 2:
                out_copies.pop(ci - 2).wait()
            idx_copies.pop(ci).wait()
            g = pltpu.async_copy(kv_hbm.at[idx_v.at[p]], rows_v.at[p], semg[p])
            if ci + 1 < chunks:
                idx_copies[ci + 1] = start_idx(ci + 1)
            g.wait()
            out_copies[ci] = pltpu.async_copy(
                rows_v.at[p],
                out_hbm.at[pl.ds(base + ci * _GATHER_CHUNK, _GATHER_CHUNK)],
                semo[p])
        for ci in sorted(out_copies):
            out_copies[ci].wait()

    return gather_kernel(kv_flat, idx_rm)


def kernel(coords, x, qkv_w, qkv_b, proj_w, proj_b, off1_w, off1_b,
           off2_w, off2_b, pos1_w, pos1_b, pos2_w, pos2_b):
    B, N, C = x.shape
    P = pos1_w.shape[0]
    nblocks = N // M
    grid = (nblocks,)

    qkv_wT = qkv_w.T
    off1_wT = off1_w.T
    off2_wT = off2_w.T
    eyeK = jnp.eye(K, dtype=jnp.float32)
    w1 = jnp.kron(eyeK, pos1_w.T)
    b1 = jnp.tile(pos1_b, K).reshape(1, K * P)
    w2s = jnp.stack([jnp.kron(eyeK, pos2_w[h].reshape(P, 1)) for h in range(H)])
    proj_wT = proj_w.T
    d = C // H
    rl_list = []
    for k in range(K):
        mk = jnp.zeros((C, H * K), jnp.float32)
        for h in range(H):
            mk = mk.at[d * h:d * (h + 1), h * K + k].set(1.0)
        rl_list.append(mk)
    rl = jnp.stack(rl_list)

    row_spec = lambda width: pl.BlockSpec((M, width), lambda i: (i, 0))
    full_spec = lambda a, b_: pl.BlockSpec((a, b_), lambda i: (0, 0))

    prep_call = pl.pallas_call(
        functools.partial(_prep_body, N),
        grid=grid,
        in_specs=[
            full_spec(3, N),
            row_spec(3),
            row_spec(C),
            full_spec(C, 3 * C),
            full_spec(1, 3 * C),
            full_spec(C, C // 2),
            full_spec(1, C // 2),
            full_spec(C // 2, 3 * K),
            full_spec(1, 3 * K),
            full_spec(3 * K, K * P),
            full_spec(1, K * P),
            pl.BlockSpec((H, K * P, K), lambda i: (0, 0, 0)),
            full_spec(1, H),
        ],
        out_specs=[
            row_spec(C),
            row_spec(2 * C),
            pl.BlockSpec((1, K, M), lambda i: (i, 0, 0)),
            row_spec(H * K),
        ],
        out_shape=[
            jax.ShapeDtypeStruct((N, C), jnp.float32),
            jax.ShapeDtypeStruct((N, 2 * C), jnp.float32),
            jax.ShapeDtypeStruct((nblocks, K, M), jnp.int32),
            jax.ShapeDtypeStruct((N, H * K), jnp.float32),
        ],
    )

    attn_call = pl.pallas_call(
        _attn_body,
        grid=grid,
        in_specs=[
            row_spec(C),
            pl.BlockSpec((1, K, M, 2 * C), lambda i: (i, 0, 0, 0)),
            row_spec(H * K),
            pl.BlockSpec((K, C, H * K), lambda i: (0, 0, 0)),
            full_spec(C, C),
            full_spec(1, C),
        ],
        out_specs=row_spec(C),
        out_shape=jax.ShapeDtypeStruct((N, C), jnp.float32),
    )

    outs = []
    for b in range(B):
        q_f, kv_f, idx_f, bias_f = prep_call(
            coords[b].T, coords[b], x[b], qkv_wT, qkv_b.reshape(1, -1),
            off1_wT, off1_b.reshape(1, -1), off2_wT, off2_b.reshape(1, -1),
            w1, b1, w2s, pos2_b.reshape(1, -1))
        sampled = _gather_rows(kv_f, idx_f.reshape(nblocks * K, M))
        outs.append(attn_call(q_f, sampled.reshape(nblocks, K, M, 2 * C), bias_f,
                              rl, proj_wT, proj_b.reshape(1, -1)))
    return jnp.stack(outs)


# speedup vs baseline: 1.0426x; 1.0426x over previous
import functools

import jax
import jax.numpy as jnp
from jax import lax
from jax.experimental import pallas as pl
from jax.experimental.pallas import tpu as pltpu
from jax.experimental.pallas import tpu_sc as plsc

H = 3
K = 16
OFFSET_SCALE = 10.0

M = 256

_SC_CORES = 2
_SC_SUBCORES = 16
_GATHER_CHUNK = 128

_INV_SQRT2 = 0.7071067811865476


def _gelu_exact(v):
    return 0.5 * v * (1.0 + lax.erf(v * _INV_SQRT2))


def _prep_body(n, coordsT_ref, coords_ref, x_ref, qkvwT_ref, qkvb_ref,
               off1T_ref, off1b_ref, off2T_ref, off2b_ref, w1_ref, b1_ref,
               w2s_ref, pos2b_ref, q_ref, kv_ref, idx_ref, bias_ref):
    xb = x_ref[...]
    qkv = jnp.dot(xb, qkvwT_ref[...], preferred_element_type=jnp.float32)
    qkv = qkv + qkvb_ref[...]
    c = xb.shape[1]
    q_ref[...] = qkv[:, :c]
    kv_ref[...] = qkv[:, c:]

    hid = jnp.dot(xb, off1T_ref[...], preferred_element_type=jnp.float32)
    hid = _gelu_exact(hid + off1b_ref[...])
    offs = jnp.dot(hid, off2T_ref[...], preferred_element_type=jnp.float32)
    offs = offs + off2b_ref[...]

    ph = jnp.dot(offs, w1_ref[...], preferred_element_type=jnp.float32)
    ph = _gelu_exact(ph + b1_ref[...])
    for h in range(H):
        bh = jnp.dot(ph, w2s_ref[h], preferred_element_type=jnp.float32)
        bias_ref[:, K * h:K * (h + 1)] = bh + pos2b_ref[:, h:h + 1]

    ct = coordsT_ref[...]
    cx, cy, cz = ct[0:1, :], ct[1:2, :], ct[2:3, :]
    cb = coords_ref[...]
    iota_f = lax.broadcasted_iota(jnp.int32, (xb.shape[0], n), 1).astype(jnp.float32)
    for k in range(K):
        spx = cb[:, 0:1] + offs[:, 3 * k:3 * k + 1] * OFFSET_SCALE
        spy = cb[:, 1:2] + offs[:, 3 * k + 1:3 * k + 2] * OFFSET_SCALE
        spz = cb[:, 2:3] + offs[:, 3 * k + 2:3 * k + 3] * OFFSET_SCALE
        dx = spx - cx
        dy = spy - cy
        dz = spz - cz
        d2 = dx * dx
        d2 = d2 + dy * dy
        d2 = d2 + dz * dz
        dmin = jnp.min(d2, axis=1, keepdims=True)
        cand = jnp.where(d2 == dmin, iota_f, float(n))
        nn = jnp.min(cand, axis=1, keepdims=True)
        idx_ref[:, k:k + 1] = nn.astype(jnp.int32)


def _attn_body(q_ref, s_ref, bias_ref, projT_ref, projb_ref, o_ref):
    q = q_ref[...]
    s = s_ref[...]
    c = q.shape[1]
    d = c // H
    scale = d ** (-0.5)
    outs = []
    for h in range(H):
        qh = q[:, d * h:d * (h + 1)]
        skh = s[:, :, d * h:d * (h + 1)]
        logits = (jnp.sum(qh[:, None, :] * skh, axis=2) * scale
                  + bias_ref[:, K * h:K * (h + 1)])
        mx = jnp.max(logits, axis=1, keepdims=True)
        e = jnp.exp(logits - mx)
        a = e / jnp.sum(e, axis=1, keepdims=True)
        svh = s[:, :, c + d * h:c + d * (h + 1)]
        outs.append(jnp.sum(a[:, :, None] * svh, axis=1))
    oc = jnp.concatenate(outs, axis=1)
    o_ref[...] = jnp.dot(oc, projT_ref[...], preferred_element_type=jnp.float32) + projb_ref[...]


def _gather_rows(kv_flat, flat_idx):
    rows, width = flat_idx.shape[0], kv_flat.shape[1]
    nw = _SC_CORES * _SC_SUBCORES
    per_w = rows // nw
    chunks = per_w // _GATHER_CHUNK
    mesh = plsc.VectorSubcoreMesh(core_axis_name="c", subcore_axis_name="s",
                                  num_cores=_SC_CORES, num_subcores=_SC_SUBCORES)

    @functools.partial(
        pl.kernel, mesh=mesh,
        out_type=jax.ShapeDtypeStruct((rows, width), jnp.float32),
        scratch_types=[
            pltpu.VMEM((2, _GATHER_CHUNK), jnp.int32),
            pltpu.VMEM((2, _GATHER_CHUNK, width), jnp.float32),
            [pltpu.SemaphoreType.DMA] * 2,
            [pltpu.SemaphoreType.DMA] * 2,
            [pltpu.SemaphoreType.DMA] * 2,
        ],
    )
    def gather_kernel(kv_hbm, idx_hbm, out_hbm, idx_v, rows_v, semi, semg, semo):
        wid = lax.axis_index("s") * _SC_CORES + lax.axis_index("c")
        base = wid * per_w

        def start_idx(ci):
            p = ci % 2
            return pltpu.async_copy(
                idx_hbm.at[pl.ds(base + ci * _GATHER_CHUNK, _GATHER_CHUNK)],
                idx_v.at[p], semi[p])

        idx_copies = {0: start_idx(0)}
        out_copies = {}
        for ci in range(chunks):
            p = ci % 2
            if ci >= 2:
                out_copies.pop(ci - 2).wait()
            idx_copies.pop(ci).wait()
            g = pltpu.async_copy(kv_hbm.at[idx_v.at[p]], rows_v.at[p], semg[p])
            if ci + 1 < chunks:
                idx_copies[ci + 1] = start_idx(ci + 1)
            g.wait()
            out_copies[ci] = pltpu.async_copy(
                rows_v.at[p],
                out_hbm.at[pl.ds(base + ci * _GATHER_CHUNK, _GATHER_CHUNK)],
                semo[p])
        for ci in sorted(out_copies):
            out_copies[ci].wait()

    return gather_kernel(kv_flat, flat_idx)


def kernel(coords, x, qkv_w, qkv_b, proj_w, proj_b, off1_w, off1_b,
           off2_w, off2_b, pos1_w, pos1_b, pos2_w, pos2_b):
    B, N, C = x.shape
    P = pos1_w.shape[0]
    nblocks = N // M
    grid = (nblocks,)

    qkv_wT = qkv_w.T
    off1_wT = off1_w.T
    off2_wT = off2_w.T
    eyeK = jnp.eye(K, dtype=jnp.float32)
    w1 = jnp.kron(eyeK, pos1_w.T)
    b1 = jnp.tile(pos1_b, K).reshape(1, K * P)
    w2s = jnp.stack([jnp.kron(eyeK, pos2_w[h].reshape(P, 1)) for h in range(H)])
    proj_wT = proj_w.T

    row_spec = lambda width: pl.BlockSpec((M, width), lambda i: (i, 0))
    full_spec = lambda a, b_: pl.BlockSpec((a, b_), lambda i: (0, 0))

    prep_call = pl.pallas_call(
        functools.partial(_prep_body, N),
        grid=grid,
        in_specs=[
            full_spec(3, N),
            row_spec(3),
            row_spec(C),
            full_spec(C, 3 * C),
            full_spec(1, 3 * C),
            full_spec(C, C // 2),
            full_spec(1, C // 2),
            full_spec(C // 2, 3 * K),
            full_spec(1, 3 * K),
            full_spec(3 * K, K * P),
            full_spec(1, K * P),
            pl.BlockSpec((H, K * P, K), lambda i: (0, 0, 0)),
            full_spec(1, H),
        ],
        out_specs=[
            row_spec(C),
            row_spec(2 * C),
            row_spec(K),
            row_spec(H * K),
        ],
        out_shape=[
            jax.ShapeDtypeStruct((N, C), jnp.float32),
            jax.ShapeDtypeStruct((N, 2 * C), jnp.float32),
            jax.ShapeDtypeStruct((N, K), jnp.int32),
            jax.ShapeDtypeStruct((N, H * K), jnp.float32),
        ],
    )

    attn_call = pl.pallas_call(
        _attn_body,
        grid=grid,
        in_specs=[
            row_spec(C),
            pl.BlockSpec((M, K, 2 * C), lambda i: (i, 0, 0)),
            row_spec(H * K),
            full_spec(C, C),
            full_spec(1, C),
        ],
        out_specs=row_spec(C),
        out_shape=jax.ShapeDtypeStruct((N, C), jnp.float32),
    )

    outs = []
    for b in range(B):
        q_f, kv_f, idx_f, bias_f = prep_call(
            coords[b].T, coords[b], x[b], qkv_wT, qkv_b.reshape(1, -1),
            off1_wT, off1_b.reshape(1, -1), off2_wT, off2_b.reshape(1, -1),
            w1, b1, w2s, pos2_b.reshape(1, -1))
        sampled = _gather_rows(kv_f, idx_f.reshape(N * K))
        outs.append(attn_call(q_f, sampled.reshape(N, K, 2 * C), bias_f,
                              proj_wT, proj_b.reshape(1, -1)))
    return jnp.stack(outs)
